# baseline (speedup 1.0000x reference)
import functools

import jax
import jax.numpy as jnp
from jax import lax
from jax.experimental import pallas as pl
from jax.experimental.pallas import tpu as pltpu
from jax.experimental.pallas import tpu_sc as plsc

N = 10000
F = 128
H = 256
C = 64
E = 320000

NC = 2
NS = 16
NW = NC * NS
EB = 128
NB = 160
E_PAD = NS * NB * EB
N_ACC = 10112
RPT = N_ACC // NS
SRT = N // NS
GW = 32

_mesh = plsc.VectorSubcoreMesh(core_axis_name="c", subcore_axis_name="s")
_sc_params = pltpu.CompilerParams(needs_layout_passes=False,
                                  use_tc_tiling_on_sc=False)


@functools.partial(
    pl.kernel,
    out_type=jax.ShapeDtypeStruct((NW, N_ACC), jnp.float32),
    mesh=_mesh,
    compiler_params=_sc_params,
    scratch_types=[
        pltpu.VMEM((NB // 2, EB), jnp.int32),
        pltpu.VMEM((N_ACC,), jnp.float32),
    ],
)
def _deg_kernel(dst_hbm, out_hbm, dst_v, deg_v):
    c = lax.axis_index("c")
    s = lax.axis_index("s")
    wid = c * NS + s
    pltpu.sync_copy(dst_hbm.at[s, pl.ds(c * (NB // 2), NB // 2)], dst_v)

    zero16 = jnp.zeros((16,), jnp.float32)

    def zbody(i, _):
        deg_v[pl.ds(i * 16, 16)] = zero16
        return 0

    lax.fori_loop(0, N_ACC // 16, zbody, 0)

    one16 = jnp.ones((16,), jnp.float32)

    def body(i, _):
        off = pl.multiple_of((i % 8) * 16, 16)
        idx = dst_v[i // 8, pl.ds(off, 16)]
        plsc.addupdate_scatter(deg_v, [idx], one16)
        return 0

    lax.fori_loop(0, (NB // 2) * 8, body, 0)
    pltpu.sync_copy(deg_v, out_hbm.at[wid])


def _make_agg_kernel(n_groups):
    ng_total = NC * n_groups

    @functools.partial(
        pl.kernel,
        out_type=jax.ShapeDtypeStruct((ng_total, N_ACC, GW), jnp.float32),
        mesh=_mesh,
        compiler_params=_sc_params,
        scratch_types=[
            pltpu.VMEM((NB, EB), jnp.int32),
            pltpu.VMEM((NB, EB), jnp.int32),
        ] + [pltpu.VMEM((EB, GW), jnp.float32)] * 4
          + [pltpu.VMEM_SHARED((N, GW), jnp.float32)]
          + [pltpu.VMEM_SHARED((N_ACC, GW), jnp.float32)]
          + [pltpu.SemaphoreType.DMA] * 8,
    )
    def _agg(table_hbm, src_hbm, dst_hbm, zeros_hbm, out_hbm, *rest):
        src_v, dst_v = rest[:2]
        bufs = rest[2:6]
        table = rest[6]
        acc_sh = rest[7]
        sgs = rest[8:12]
        sss = rest[12:16]

        c = lax.axis_index("c")
        s = lax.axis_index("s")
        pltpu.sync_copy(src_hbm.at[s], src_v)
        pltpu.sync_copy(dst_hbm.at[s], dst_v)

        for g in range(n_groups):
            pltpu.sync_copy(table_hbm.at[c * n_groups + g, pl.ds(s * SRT, SRT)],
                            table.at[pl.ds(s * SRT, SRT)])
            pltpu.sync_copy(zeros_hbm.at[pl.ds(s * RPT, RPT)],
                            acc_sh.at[pl.ds(s * RPT, RPT)])
            plsc.subcore_barrier()

            hbm_tbl = table_hbm.at[c * n_groups + g]
            tbls = (table, table, table, hbm_tbl)

            for j in range(4):
                pltpu.async_copy(tbls[j].at[src_v.at[j]], bufs[j], sgs[j])

            def body(i, _):
                base = 4 * i
                for j in range(4):
                    b = base + j
                    pltpu.make_async_copy(
                        tbls[j].at[src_v.at[b]], bufs[j], sgs[j]).wait()
                    pltpu.async_copy(
                        bufs[j], acc_sh.at[dst_v.at[b]], sss[j], add=True)
                for j in range(4):
                    b = base + j
                    pltpu.make_async_copy(
                        bufs[j], acc_sh.at[dst_v.at[b]], sss[j]).wait()

                    def _start_next(j=j, b=b):
                        pltpu.async_copy(
                            tbls[j].at[src_v.at[b + 4]], bufs[j], sgs[j])

                    pl.when(b + 4 < NB)(_start_next)
                return 0

            lax.fori_loop(0, NB // 4, body, 0)
            plsc.subcore_barrier()
            pltpu.sync_copy(acc_sh.at[pl.ds(s * RPT, RPT)],
                            out_hbm.at[c * n_groups + g, pl.ds(s * RPT, RPT)])

    return _agg


_agg1 = _make_agg_kernel(2)
_agg2 = _make_agg_kernel(1)


def _dinv_from_parts(degp):
    deg = jnp.sum(degp, axis=0)[:N] + 1.0
    return lax.rsqrt(deg)[:, None]


def _tc_x4_body(degp_ref, x_ref, x4_ref, dinv_ref):
    dinv = _dinv_from_parts(degp_ref[...])
    y = x_ref[...] * dinv
    for g in range(4):
        x4_ref[g] = y[:, g * GW:(g + 1) * GW]
    dinv_ref[...] = dinv


def _tc_x4(deg_parts, x):
    return pl.pallas_call(
        _tc_x4_body,
        out_shape=[jax.ShapeDtypeStruct((4, N, GW), jnp.float32),
                   jax.ShapeDtypeStruct((N, 1), jnp.float32)],
    )(deg_parts, x)


_RB = N // 5


def _tc2_body(acc_ref, x4_ref, dinv_ref, w1_ref, b1_ref, w2_ref, b2_ref, y2_ref):
    dinv = dinv_ref[...]
    agg = jnp.concatenate(
        [acc_ref[g] + x4_ref[g] for g in range(4)], axis=1)
    z = agg * dinv
    h = jnp.dot(z, w1_ref[...], preferred_element_type=jnp.float32) + b1_ref[...]
    h = jnp.maximum(h, 0.0)
    y2 = jnp.dot(h, w2_ref[...], preferred_element_type=jnp.float32) * dinv
    y2_ref[0] = y2[:, :GW]
    y2_ref[1] = y2[:, GW:]


def _tc2(acc1, x4s, dinvc, W1, b1, W2, b2):
    return pl.pallas_call(
        _tc2_body,
        grid=(N // _RB,),
        in_specs=[
            pl.BlockSpec((4, _RB, GW), lambda i: (0, i, 0)),
            pl.BlockSpec((4, _RB, GW), lambda i: (0, i, 0)),
            pl.BlockSpec((_RB, 1), lambda i: (i, 0)),
            pl.BlockSpec((F, H), lambda i: (0, 0)),
            pl.BlockSpec((1, H), lambda i: (0, 0)),
            pl.BlockSpec((H, C), lambda i: (0, 0)),
            pl.BlockSpec((1, C), lambda i: (0, 0)),
        ],
        out_specs=pl.BlockSpec((NC, _RB, GW), lambda i: (0, i, 0)),
        out_shape=jax.ShapeDtypeStruct((NC, N, GW), jnp.float32),
    )(acc1, x4s, dinvc, W1, b1, W2, b2)


def _tc3_body(acc_ref, y2_ref, dinv_ref, b2_ref, out_ref):
    s = jnp.concatenate(
        [acc_ref[g] + y2_ref[g] for g in range(NC)], axis=1)
    out_ref[...] = s * dinv_ref[...] + b2_ref[...]


def _tc3(acc2, y2g, dinvc, b2):
    return pl.pallas_call(
        _tc3_body,
        grid=(N // _RB,),
        in_specs=[
            pl.BlockSpec((NC, _RB, GW), lambda i: (0, i, 0)),
            pl.BlockSpec((NC, _RB, GW), lambda i: (0, i, 0)),
            pl.BlockSpec((_RB, 1), lambda i: (i, 0)),
            pl.BlockSpec((1, C), lambda i: (0, 0)),
        ],
        out_specs=pl.BlockSpec((_RB, C), lambda i: (i, 0)),
        out_shape=jax.ShapeDtypeStruct((N, C), jnp.float32),
    )(acc2, y2g, dinvc, b2)


def kernel(x, edge_index, W1, b1, W2, b2):
    ei = edge_index.astype(jnp.int32)
    src, dst = ei[0], ei[1]
    pad = E_PAD - E
    src_pad = jnp.concatenate([src, jnp.zeros((pad,), jnp.int32)])
    dst_pad = jnp.concatenate([dst, jnp.full((pad,), N, jnp.int32)])
    src16 = src_pad.reshape(NS, NB, EB)
    dst16 = dst_pad.reshape(NS, NB, EB)

    zeros_gw = jnp.zeros((N_ACC, GW), jnp.float32)

    deg_parts = _deg_kernel(dst16)
    x4s, dinvc = _tc_x4(deg_parts, x)
    acc1 = _agg1(x4s, src16, dst16, zeros_gw)
    y2g = _tc2(acc1, x4s, dinvc,
               W1, b1.reshape(1, H), W2, b2.reshape(1, C))
    acc2 = _agg2(y2g, src16, dst16, zeros_gw)
    return _tc3(acc2, y2g, dinvc, b2.reshape(1, C))


# speedup vs baseline: 1.0664x; 1.0664x over previous
import functools

import jax
import jax.numpy as jnp
from jax import lax
from jax.experimental import pallas as pl
from jax.experimental.pallas import tpu as pltpu
from jax.experimental.pallas import tpu_sc as plsc

N = 10000
F = 128
H = 256
C = 64
E = 320000

NC = 2
NS = 16
NW = NC * NS
EB = 128
NB = 160
E_PAD = NS * NB * EB
N_ACC = 10112
RPT = N_ACC // NS
SRT = N // NS
GW = 32

_mesh = plsc.VectorSubcoreMesh(core_axis_name="c", subcore_axis_name="s")
_sc_params = pltpu.CompilerParams(needs_layout_passes=False,
                                  use_tc_tiling_on_sc=False)


@functools.partial(
    pl.kernel,
    out_type=jax.ShapeDtypeStruct((NW, N_ACC), jnp.float32),
    mesh=_mesh,
    compiler_params=_sc_params,
    scratch_types=[
        pltpu.VMEM((NB // 2, EB), jnp.int32),
        pltpu.VMEM((N_ACC,), jnp.float32),
    ],
)
def _deg_kernel(dst_hbm, out_hbm, dst_v, deg_v):
    c = lax.axis_index("c")
    s = lax.axis_index("s")
    wid = c * NS + s
    pltpu.sync_copy(dst_hbm.at[s, pl.ds(c * (NB // 2), NB // 2)], dst_v)

    zero16 = jnp.zeros((16,), jnp.float32)

    def zbody(i, _):
        deg_v[pl.ds(i * 16, 16)] = zero16
        return 0

    lax.fori_loop(0, N_ACC // 16, zbody, 0)

    one16 = jnp.ones((16,), jnp.float32)

    def body(i, _):
        off = pl.multiple_of((i % 8) * 16, 16)
        idx = dst_v[i // 8, pl.ds(off, 16)]
        plsc.addupdate_scatter(deg_v, [idx], one16)
        return 0

    lax.fori_loop(0, (NB // 2) * 8, body, 0)
    pltpu.sync_copy(deg_v, out_hbm.at[wid])


def _make_agg_kernel(n_groups):
    ng_total = NC * n_groups

    @functools.partial(
        pl.kernel,
        out_type=jax.ShapeDtypeStruct((ng_total, N_ACC, GW), jnp.float32),
        mesh=_mesh,
        compiler_params=_sc_params,
        scratch_types=[
            pltpu.VMEM((NB, EB), jnp.int32),
            pltpu.VMEM((NB, EB), jnp.int32),
        ] + [pltpu.VMEM((EB, GW), jnp.float32)] * 4
          + [pltpu.VMEM_SHARED((N, GW), jnp.float32)]
          + [pltpu.VMEM_SHARED((N_ACC, GW), jnp.float32)]
          + [pltpu.SemaphoreType.DMA] * 8,
    )
    def _agg(table_hbm, src_hbm, dst_hbm, zeros_hbm, out_hbm, *rest):
        src_v, dst_v = rest[:2]
        bufs = rest[2:6]
        table = rest[6]
        acc_sh = rest[7]
        sgs = rest[8:12]
        sss = rest[12:16]

        c = lax.axis_index("c")
        s = lax.axis_index("s")
        pltpu.sync_copy(src_hbm.at[s], src_v)
        pltpu.sync_copy(dst_hbm.at[s], dst_v)

        for g in range(n_groups):
            pltpu.sync_copy(table_hbm.at[c * n_groups + g, pl.ds(s * SRT, SRT)],
                            table.at[pl.ds(s * SRT, SRT)])
            pltpu.sync_copy(zeros_hbm.at[pl.ds(s * RPT, RPT)],
                            acc_sh.at[pl.ds(s * RPT, RPT)])
            plsc.subcore_barrier()

            tbls = (table, table, table, table)

            for j in range(4):
                pltpu.async_copy(tbls[j].at[src_v.at[j]], bufs[j], sgs[j])

            def body(i, _):
                base = 4 * i
                for j in range(4):
                    b = base + j
                    pltpu.make_async_copy(
                        tbls[j].at[src_v.at[b]], bufs[j], sgs[j]).wait()
                    pltpu.async_copy(
                        bufs[j], acc_sh.at[dst_v.at[b]], sss[j], add=True)
                for j in range(4):
                    b = base + j
                    pltpu.make_async_copy(
                        bufs[j], acc_sh.at[dst_v.at[b]], sss[j]).wait()

                    def _start_next(j=j, b=b):
                        pltpu.async_copy(
                            tbls[j].at[src_v.at[b + 4]], bufs[j], sgs[j])

                    pl.when(b + 4 < NB)(_start_next)
                return 0

            lax.fori_loop(0, NB // 4, body, 0)
            plsc.subcore_barrier()
            pltpu.sync_copy(acc_sh.at[pl.ds(s * RPT, RPT)],
                            out_hbm.at[c * n_groups + g, pl.ds(s * RPT, RPT)])

    return _agg


_agg1 = _make_agg_kernel(2)
_agg2 = _make_agg_kernel(1)


def _dinv_from_parts(degp):
    deg = jnp.sum(degp, axis=0)[:N] + 1.0
    return lax.rsqrt(deg)[:, None]


def _tc_x4_body(degp_ref, x_ref, x4_ref, dinv_ref):
    dinv = _dinv_from_parts(degp_ref[...])
    y = x_ref[...] * dinv
    for g in range(4):
        x4_ref[g] = y[:, g * GW:(g + 1) * GW]
    dinv_ref[...] = dinv


def _tc_x4(deg_parts, x):
    return pl.pallas_call(
        _tc_x4_body,
        out_shape=[jax.ShapeDtypeStruct((4, N, GW), jnp.float32),
                   jax.ShapeDtypeStruct((N, 1), jnp.float32)],
    )(deg_parts, x)


_RB = N // 5


def _tc2_body(acc_ref, x4_ref, dinv_ref, w1_ref, b1_ref, w2_ref, b2_ref, y2_ref):
    dinv = dinv_ref[...]
    agg = jnp.concatenate(
        [acc_ref[g] + x4_ref[g] for g in range(4)], axis=1)
    z = agg * dinv
    h = jnp.dot(z, w1_ref[...], preferred_element_type=jnp.float32) + b1_ref[...]
    h = jnp.maximum(h, 0.0)
    y2 = jnp.dot(h, w2_ref[...], preferred_element_type=jnp.float32) * dinv
    y2_ref[0] = y2[:, :GW]
    y2_ref[1] = y2[:, GW:]


def _tc2(acc1, x4s, dinvc, W1, b1, W2, b2):
    return pl.pallas_call(
        _tc2_body,
        grid=(N // _RB,),
        in_specs=[
            pl.BlockSpec((4, _RB, GW), lambda i: (0, i, 0)),
            pl.BlockSpec((4, _RB, GW), lambda i: (0, i, 0)),
            pl.BlockSpec((_RB, 1), lambda i: (i, 0)),
            pl.BlockSpec((F, H), lambda i: (0, 0)),
            pl.BlockSpec((1, H), lambda i: (0, 0)),
            pl.BlockSpec((H, C), lambda i: (0, 0)),
            pl.BlockSpec((1, C), lambda i: (0, 0)),
        ],
        out_specs=pl.BlockSpec((NC, _RB, GW), lambda i: (0, i, 0)),
        out_shape=jax.ShapeDtypeStruct((NC, N, GW), jnp.float32),
    )(acc1, x4s, dinvc, W1, b1, W2, b2)


def _tc3_body(acc_ref, y2_ref, dinv_ref, b2_ref, out_ref):
    s = jnp.concatenate(
        [acc_ref[g] + y2_ref[g] for g in range(NC)], axis=1)
    out_ref[...] = s * dinv_ref[...] + b2_ref[...]


def _tc3(acc2, y2g, dinvc, b2):
    return pl.pallas_call(
        _tc3_body,
        grid=(N // _RB,),
        in_specs=[
            pl.BlockSpec((NC, _RB, GW), lambda i: (0, i, 0)),
            pl.BlockSpec((NC, _RB, GW), lambda i: (0, i, 0)),
            pl.BlockSpec((_RB, 1), lambda i: (i, 0)),
            pl.BlockSpec((1, C), lambda i: (0, 0)),
        ],
        out_specs=pl.BlockSpec((_RB, C), lambda i: (i, 0)),
        out_shape=jax.ShapeDtypeStruct((N, C), jnp.float32),
    )(acc2, y2g, dinvc, b2)


def kernel(x, edge_index, W1, b1, W2, b2):
    ei = edge_index.astype(jnp.int32)
    src, dst = ei[0], ei[1]
    pad = E_PAD - E
    src_pad = jnp.concatenate([src, jnp.zeros((pad,), jnp.int32)])
    dst_pad = jnp.concatenate([dst, jnp.full((pad,), N, jnp.int32)])
    src16 = src_pad.reshape(NS, NB, EB)
    dst16 = dst_pad.reshape(NS, NB, EB)

    zeros_gw = jnp.zeros((N_ACC, GW), jnp.float32)

    deg_parts = _deg_kernel(dst16)
    x4s, dinvc = _tc_x4(deg_parts, x)
    acc1 = _agg1(x4s, src16, dst16, zeros_gw)
    y2g = _tc2(acc1, x4s, dinvc,
               W1, b1.reshape(1, H), W2, b2.reshape(1, C))
    acc2 = _agg2(y2g, src16, dst16, zeros_gw)
    return _tc3(acc2, y2g, dinvc, b2.reshape(1, C))


# speedup vs baseline: 1.1142x; 1.0448x over previous
import functools

import jax
import jax.numpy as jnp
from jax import lax
from jax.experimental import pallas as pl
from jax.experimental.pallas import tpu as pltpu
from jax.experimental.pallas import tpu_sc as plsc

N = 10000
F = 128
H = 256
C = 64
E = 320000

NC = 2
NS = 16
NW = NC * NS
EB = 128
NB = 160
E_PAD = NS * NB * EB
N_ACC = 10112
RPT = N_ACC // NS
SRT = N // NS
GW = 32

_mesh = plsc.VectorSubcoreMesh(core_axis_name="c", subcore_axis_name="s")
_sc_params = pltpu.CompilerParams(needs_layout_passes=False,
                                  use_tc_tiling_on_sc=False)


@functools.partial(
    pl.kernel,
    out_type=jax.ShapeDtypeStruct((NW, N_ACC), jnp.float32),
    mesh=_mesh,
    compiler_params=_sc_params,
    scratch_types=[
        pltpu.VMEM((NB // 2, EB), jnp.int32),
        pltpu.VMEM((N_ACC,), jnp.float32),
    ],
)
def _deg_kernel(dst_hbm, out_hbm, dst_v, deg_v):
    c = lax.axis_index("c")
    s = lax.axis_index("s")
    wid = c * NS + s
    pltpu.sync_copy(dst_hbm.at[s, pl.ds(c * (NB // 2), NB // 2)], dst_v)

    zero16 = jnp.zeros((16,), jnp.float32)

    def zbody(i, _):
        deg_v[pl.ds(i * 16, 16)] = zero16
        return 0

    lax.fori_loop(0, N_ACC // 16, zbody, 0)

    one16 = jnp.ones((16,), jnp.float32)

    def body(i, _):
        off = pl.multiple_of((i % 8) * 16, 16)
        idx = dst_v[i // 8, pl.ds(off, 16)]
        plsc.addupdate_scatter(deg_v, [idx], one16)
        return 0

    lax.fori_loop(0, (NB // 2) * 8, body, 0)
    pltpu.sync_copy(deg_v, out_hbm.at[wid])


def _make_agg_kernel(n_groups):
    ng_total = NC * n_groups

    @functools.partial(
        pl.kernel,
        out_type=jax.ShapeDtypeStruct((ng_total, N_ACC, GW), jnp.float32),
        mesh=_mesh,
        compiler_params=_sc_params,
        scratch_types=[
            pltpu.VMEM((NB, EB), jnp.int32),
            pltpu.VMEM((NB, EB), jnp.int32),
        ] + [pltpu.VMEM((EB, GW), jnp.float32)] * 8
          + [pltpu.VMEM_SHARED((N, GW), jnp.float32)]
          + [pltpu.VMEM_SHARED((N_ACC, GW), jnp.float32)]
          + [pltpu.SemaphoreType.DMA] * 16,
    )
    def _agg(table_hbm, src_hbm, dst_hbm, zeros_hbm, out_hbm, *rest):
        src_v, dst_v = rest[:2]
        bufs = rest[2:10]
        table = rest[10]
        acc_sh = rest[11]
        sgs = rest[12:20]
        sss = rest[20:28]

        c = lax.axis_index("c")
        s = lax.axis_index("s")
        pltpu.sync_copy(src_hbm.at[s], src_v)
        pltpu.sync_copy(dst_hbm.at[s], dst_v)

        for g in range(n_groups):
            pltpu.sync_copy(table_hbm.at[c * n_groups + g, pl.ds(s * SRT, SRT)],
                            table.at[pl.ds(s * SRT, SRT)])
            pltpu.sync_copy(zeros_hbm.at[pl.ds(s * RPT, RPT)],
                            acc_sh.at[pl.ds(s * RPT, RPT)])
            plsc.subcore_barrier()

            tbls = (table,) * 8

            for j in range(8):
                pltpu.async_copy(tbls[j].at[src_v.at[j]], bufs[j], sgs[j])

            def body(i, _):
                base = 8 * i
                for j in range(8):
                    b = base + j
                    pltpu.make_async_copy(
                        tbls[j].at[src_v.at[b]], bufs[j], sgs[j]).wait()
                    pltpu.async_copy(
                        bufs[j], acc_sh.at[dst_v.at[b]], sss[j], add=True)
                for j in range(8):
                    b = base + j
                    pltpu.make_async_copy(
                        bufs[j], acc_sh.at[dst_v.at[b]], sss[j]).wait()

                    def _start_next(j=j, b=b):
                        pltpu.async_copy(
                            tbls[j].at[src_v.at[b + 8]], bufs[j], sgs[j])

                    pl.when(b + 8 < NB)(_start_next)
                return 0

            lax.fori_loop(0, NB // 8, body, 0)
            plsc.subcore_barrier()
            pltpu.sync_copy(acc_sh.at[pl.ds(s * RPT, RPT)],
                            out_hbm.at[c * n_groups + g, pl.ds(s * RPT, RPT)])

    return _agg


_agg1 = _make_agg_kernel(2)
_agg2 = _make_agg_kernel(1)


def _dinv_from_parts(degp):
    deg = jnp.sum(degp, axis=0)[:N] + 1.0
    return lax.rsqrt(deg)[:, None]


def _tc_x4_body(degp_ref, x_ref, x4_ref, dinv_ref):
    dinv = _dinv_from_parts(degp_ref[...])
    y = x_ref[...] * dinv
    for g in range(4):
        x4_ref[g] = y[:, g * GW:(g + 1) * GW]
    dinv_ref[...] = dinv


def _tc_x4(deg_parts, x):
    return pl.pallas_call(
        _tc_x4_body,
        out_shape=[jax.ShapeDtypeStruct((4, N, GW), jnp.float32),
                   jax.ShapeDtypeStruct((N, 1), jnp.float32)],
    )(deg_parts, x)


_RB = N // 5


def _tc2_body(acc_ref, x4_ref, dinv_ref, w1_ref, b1_ref, w2_ref, b2_ref, y2_ref):
    dinv = dinv_ref[...]
    agg = jnp.concatenate(
        [acc_ref[g] + x4_ref[g] for g in range(4)], axis=1)
    z = agg * dinv
    h = jnp.dot(z, w1_ref[...], preferred_element_type=jnp.float32) + b1_ref[...]
    h = jnp.maximum(h, 0.0)
    y2 = jnp.dot(h, w2_ref[...], preferred_element_type=jnp.float32) * dinv
    y2_ref[0] = y2[:, :GW]
    y2_ref[1] = y2[:, GW:]


def _tc2(acc1, x4s, dinvc, W1, b1, W2, b2):
    return pl.pallas_call(
        _tc2_body,
        grid=(N // _RB,),
        in_specs=[
            pl.BlockSpec((4, _RB, GW), lambda i: (0, i, 0)),
            pl.BlockSpec((4, _RB, GW), lambda i: (0, i, 0)),
            pl.BlockSpec((_RB, 1), lambda i: (i, 0)),
            pl.BlockSpec((F, H), lambda i: (0, 0)),
            pl.BlockSpec((1, H), lambda i: (0, 0)),
            pl.BlockSpec((H, C), lambda i: (0, 0)),
            pl.BlockSpec((1, C), lambda i: (0, 0)),
        ],
        out_specs=pl.BlockSpec((NC, _RB, GW), lambda i: (0, i, 0)),
        out_shape=jax.ShapeDtypeStruct((NC, N, GW), jnp.float32),
    )(acc1, x4s, dinvc, W1, b1, W2, b2)


def _tc3_body(acc_ref, y2_ref, dinv_ref, b2_ref, out_ref):
    s = jnp.concatenate(
        [acc_ref[g] + y2_ref[g] for g in range(NC)], axis=1)
    out_ref[...] = s * dinv_ref[...] + b2_ref[...]


def _tc3(acc2, y2g, dinvc, b2):
    return pl.pallas_call(
        _tc3_body,
        grid=(N // _RB,),
        in_specs=[
            pl.BlockSpec((NC, _RB, GW), lambda i: (0, i, 0)),
            pl.BlockSpec((NC, _RB, GW), lambda i: (0, i, 0)),
            pl.BlockSpec((_RB, 1), lambda i: (i, 0)),
            pl.BlockSpec((1, C), lambda i: (0, 0)),
        ],
        out_specs=pl.BlockSpec((_RB, C), lambda i: (i, 0)),
        out_shape=jax.ShapeDtypeStruct((N, C), jnp.float32),
    )(acc2, y2g, dinvc, b2)


def kernel(x, edge_index, W1, b1, W2, b2):
    ei = edge_index.astype(jnp.int32)
    src, dst = ei[0], ei[1]
    pad = E_PAD - E
    src_pad = jnp.concatenate([src, jnp.zeros((pad,), jnp.int32)])
    dst_pad = jnp.concatenate([dst, jnp.full((pad,), N, jnp.int32)])
    src16 = src_pad.reshape(NS, NB, EB)
    dst16 = dst_pad.reshape(NS, NB, EB)

    zeros_gw = jnp.zeros((N_ACC, GW), jnp.float32)

    deg_parts = _deg_kernel(dst16)
    x4s, dinvc = _tc_x4(deg_parts, x)
    acc1 = _agg1(x4s, src16, dst16, zeros_gw)
    y2g = _tc2(acc1, x4s, dinvc,
               W1, b1.reshape(1, H), W2, b2.reshape(1, C))
    acc2 = _agg2(y2g, src16, dst16, zeros_gw)
    return _tc3(acc2, y2g, dinvc, b2.reshape(1, C))
